# Initial kernel scaffold; baseline (speedup 1.0000x reference)
import jax
import jax.numpy as jnp
from jax.experimental import pallas as pl


def kernel(x, W1, b1, W2, b2, W3, b3, orders, perms):
    raise NotImplementedError("write your pallas kernel here")


# speedup vs baseline: 6.5993x; 6.5993x over previous
import functools

import jax
import jax.numpy as jnp
import numpy as np
from jax.experimental import pallas as pl
from jax.experimental.pallas import tpu as pltpu

_B = 262144
_DIM = 6
_HALF = 3
_K = 40
_H = 128
_NL = 8
_PI = float(np.pi)
_BR = 2048

_ATAN_C = (0.9999999987329571, -0.3333329490271314, 0.19998530422323615,
           -0.14264510232090435, 0.10954998354223984, -0.0841450751516909,
           0.05818360636033609, -0.03143228778537418, 0.011064244656339386,
           -0.0018295627827675104)


def _atan(t):
    a = jnp.abs(t)
    big = a > 1.0
    r = jnp.where(big, 1.0 / a, a)
    r2 = r * r
    p = jnp.full_like(r2, _ATAN_C[-1])
    for c in _ATAN_C[-2::-1]:
        p = p * r2 + c
    at = p * r
    at = jnp.where(big, (_PI / 2.0) - at, at)
    return jnp.where(t < 0.0, -at, at)


def _fused_body(x_ref, W1_ref, b1_ref, W2_ref, b2_ref, W3_ref, b3_ref,
                E_ref, G_ref, M_ref, o_ref):
    f32 = jnp.float32
    xb = x_ref[...]
    xp = jnp.clip(xb[:, :_HALF], -1.0, 1.0)
    v = xb[:, _HALF:] * (1.0 / _PI)
    n2 = jnp.sum(v * v, axis=1, keepdims=True)
    den = jax.lax.rsqrt(jnp.maximum(1.0 - n2, 1e-12))
    yc = _atan(v * den) * (2.0 / _PI)
    yc = jnp.where(n2 < 1.0, yc, 0.0)
    y = jnp.concatenate([xp, yc], axis=1)
    z = jnp.dot(y, M_ref[0], preferred_element_type=f32)

    i120 = jax.lax.broadcasted_iota(jnp.int32, (1, _HALF * _K), 1).astype(f32)
    klocal = i120 - float(_K) * jnp.floor(i120 * (1.0 / _K))

    for i in range(_NL):
        xid = z[:, :_HALF]
        xt = z[:, _HALF:]
        h = jnp.maximum(
            jnp.dot(xid, W1_ref[i], preferred_element_type=f32) + b1_ref[i], 0.0)
        h = jnp.maximum(
            jnp.dot(h, W2_ref[i], preferred_element_type=f32) + b2_ref[i], 0.0)
        th = jnp.dot(h, W3_ref[i], preferred_element_type=f32) + b3_ref[i]
        th = jnp.clip(th, -60.0, 60.0)
        e = jnp.exp(th)
        pos = jnp.clip((xt + 1.0) * (0.5 * _K), 0.0, float(_K))
        posf = jnp.dot(pos, E_ref[...], preferred_element_type=f32)
        w = jnp.clip(posf - klocal, 0.0, 1.0)
        num = jnp.dot(e * w, G_ref[...], preferred_element_type=f32)
        dn = jnp.dot(e, G_ref[...], preferred_element_type=f32)
        yt = (num / dn) * 2.0 - 1.0
        yp = jnp.concatenate([xid, yt], axis=1)
        z = jnp.dot(yp, M_ref[i + 1], preferred_element_type=f32)
    o_ref[...] = z


@jax.jit
def kernel(x, W1, b1, W2, b2, W3, b3, orders, perms):
    f32 = jnp.float32
    eye = jnp.eye(_DIM, dtype=f32)
    inv = jnp.argsort(orders, axis=1)
    perm_mats = []
    perm_mats.append(jnp.transpose(eye[orders[0]]))
    for i in range(_NL - 1):
        c = inv[i][perms[i]][orders[i + 1]]
        perm_mats.append(jnp.transpose(eye[c]))
    perm_mats.append(jnp.transpose(eye[inv[_NL - 1]]))
    M = jnp.stack(perm_mats)

    gidx = np.repeat(np.arange(_HALF), _K)
    E = jnp.asarray(np.eye(_HALF, dtype=np.float32)[:, gidx])
    G = jnp.asarray(np.eye(_HALF, dtype=np.float32)[gidx, :])

    grid = (_B // _BR,)
    out = pl.pallas_call(
        _fused_body,
        grid=grid,
        in_specs=[
            pl.BlockSpec((_BR, _DIM), lambda i: (i, 0)),
            pl.BlockSpec((_NL, _HALF, _H), lambda i: (0, 0, 0)),
            pl.BlockSpec((_NL, _H), lambda i: (0, 0)),
            pl.BlockSpec((_NL, _H, _H), lambda i: (0, 0, 0)),
            pl.BlockSpec((_NL, _H), lambda i: (0, 0)),
            pl.BlockSpec((_NL, _H, _HALF * _K), lambda i: (0, 0, 0)),
            pl.BlockSpec((_NL, _HALF * _K), lambda i: (0, 0)),
            pl.BlockSpec((_HALF, _HALF * _K), lambda i: (0, 0)),
            pl.BlockSpec((_HALF * _K, _HALF), lambda i: (0, 0)),
            pl.BlockSpec((_NL + 1, _DIM, _DIM), lambda i: (0, 0, 0)),
        ],
        out_specs=pl.BlockSpec((_BR, _DIM), lambda i: (i, 0)),
        out_shape=jax.ShapeDtypeStruct((_B, _DIM), f32),
        compiler_params=pltpu.CompilerParams(
            dimension_semantics=("parallel",)),
    )(x, W1, b1, W2, b2, W3, b3, E, G, M)
    return out


# speedup vs baseline: 9.7205x; 1.4729x over previous
import jax
import jax.numpy as jnp
import numpy as np
from jax.experimental import pallas as pl
from jax.experimental.pallas import tpu as pltpu

_B = 262144
_DIM = 6
_HALF = 3
_K = 40
_H = 128
_NL = 8
_PI = float(np.pi)
_BR = 2048

_ATAN_C = (0.9999999987329571, -0.3333329490271314, 0.19998530422323615,
           -0.14264510232090435, 0.10954998354223984, -0.0841450751516909,
           0.05818360636033609, -0.03143228778537418, 0.011064244656339386,
           -0.0018295627827675104)


def _atan(t):
    a = jnp.abs(t)
    big = a > 1.0
    r = jnp.where(big, 1.0 / a, a)
    r2 = r * r
    p = jnp.full_like(r2, _ATAN_C[-1])
    for c in _ATAN_C[-2::-1]:
        p = p * r2 + c
    at = p * r
    at = jnp.where(big, (_PI / 2.0) - at, at)
    return jnp.where(t < 0.0, -at, at)


def _fused_body(x_ref, Aeff_ref, b1_ref, W2T_ref, b2_ref, W3T_ref, b3_ref,
                ck_ref, R4_ref, M8_ref, o_ref):
    f32 = jnp.float32
    xT = jnp.transpose(x_ref[...])
    xpT = jnp.clip(xT[:_HALF], -1.0, 1.0)
    vT = xT[_HALF:] * (1.0 / _PI)
    n2 = jnp.sum(vT * vT, axis=0, keepdims=True)
    den = jax.lax.rsqrt(jnp.maximum(1.0 - n2, 1e-12))
    ycT = _atan(vT * den) * (2.0 / _PI)
    ycT = jnp.where(n2 < 1.0, ycT, 0.0)
    ypT = jnp.concatenate([xpT, ycT], axis=0)

    for i in range(_NL):
        a = jnp.dot(Aeff_ref[i], ypT, preferred_element_type=f32)
        h = jnp.maximum(a[:_H] + b1_ref[i], 0.0)
        h = jnp.maximum(
            jnp.dot(W2T_ref[i], h, preferred_element_type=f32) + b2_ref[i],
            0.0)
        th = jnp.dot(W3T_ref[i], h, preferred_element_type=f32) + b3_ref[i]
        e = jnp.exp(jnp.clip(th, -60.0, 60.0))
        w = jnp.clip(a[_H:2 * _H] + ck_ref[...], 0.0, 1.0)
        ewe = jnp.concatenate([e * w, e], axis=0)
        f4 = jnp.dot(R4_ref[...], ewe, preferred_element_type=f32)
        yt = f4[:_HALF] / f4[_HALF:]
        ypT = jnp.concatenate([a[2 * _H:2 * _H + _HALF], yt], axis=0)
    o_ref[...] = jax.lax.dot_general(
        ypT, M8_ref[...], (((0,), (0,)), ((), ())),
        preferred_element_type=f32)


@jax.jit
def kernel(x, W1, b1, W2, b2, W3, b3, orders, perms):
    f32 = jnp.float32
    eye6 = jnp.eye(_DIM, dtype=f32)
    inv = jnp.argsort(orders, axis=1)
    perm_mats = [jnp.transpose(eye6[orders[0]])]
    for i in range(_NL - 1):
        c = inv[i][perms[i]][orders[i + 1]]
        perm_mats.append(jnp.transpose(eye6[c]))
    perm_mats.append(jnp.transpose(eye6[inv[_NL - 1]]))

    gidx = np.repeat(np.arange(_HALF), _K)
    klocal = np.tile(np.arange(_K), _HALF).astype(np.float32)
    E20 = np.zeros((_HALF, _H), dtype=np.float32)
    E20[gidx, np.arange(_HALF * _K)] = 20.0
    E20 = jnp.asarray(E20)
    Aeff_list = []
    for i in range(_NL):
        M = perm_mats[i]
        blk1 = jnp.transpose(M[:, :_HALF] @ W1[i])
        blk2 = jnp.transpose(M[:, _HALF:] @ E20)
        blk3 = jnp.transpose(M[:, :_HALF])
        Aeff_list.append(jnp.concatenate(
            [blk1, blk2, blk3, jnp.zeros((5, _DIM), f32)], axis=0))
    Aeff = jnp.stack(Aeff_list)

    b1c = b1[:, :, None]
    b2c = b2[:, :, None]
    b3c = jnp.pad(b3, ((0, 0), (0, _H - _HALF * _K)))[:, :, None]
    ck = np.full((_H, 1), -1.0, dtype=np.float32)
    ck[:_HALF * _K, 0] = 20.0 - klocal
    ck = jnp.asarray(ck)

    W2T = jnp.transpose(W2, (0, 2, 1))
    W3T = jnp.transpose(
        jnp.pad(W3, ((0, 0), (0, 0), (0, _H - _HALF * _K))), (0, 2, 1))

    R4 = np.zeros((_DIM, 2 * _H), dtype=np.float32)
    R4[gidx, np.arange(_HALF * _K)] = 2.0
    R4[gidx, _H + np.arange(_HALF * _K)] = -1.0
    R4[_HALF + gidx, _H + np.arange(_HALF * _K)] = 1.0
    R4 = jnp.asarray(R4)

    M8 = perm_mats[_NL]

    grid = (_B // _BR,)
    out = pl.pallas_call(
        _fused_body,
        grid=grid,
        in_specs=[
            pl.BlockSpec((_BR, _DIM), lambda i: (i, 0)),
            pl.BlockSpec((_NL, 264, _DIM), lambda i: (0, 0, 0)),
            pl.BlockSpec((_NL, _H, 1), lambda i: (0, 0, 0)),
            pl.BlockSpec((_NL, _H, _H), lambda i: (0, 0, 0)),
            pl.BlockSpec((_NL, _H, 1), lambda i: (0, 0, 0)),
            pl.BlockSpec((_NL, _H, _H), lambda i: (0, 0, 0)),
            pl.BlockSpec((_NL, _H, 1), lambda i: (0, 0, 0)),
            pl.BlockSpec((_H, 1), lambda i: (0, 0)),
            pl.BlockSpec((_DIM, 2 * _H), lambda i: (0, 0)),
            pl.BlockSpec((_DIM, _DIM), lambda i: (0, 0)),
        ],
        out_specs=pl.BlockSpec((_BR, _DIM), lambda i: (i, 0)),
        out_shape=jax.ShapeDtypeStruct((_B, _DIM), f32),
        compiler_params=pltpu.CompilerParams(
            dimension_semantics=("parallel",)),
    )(x, Aeff, b1c, W2T, b2c, W3T, b3c, ck, R4, M8)
    return out


# speedup vs baseline: 10.1222x; 1.0413x over previous
import jax
import jax.numpy as jnp
import numpy as np
from jax.experimental import pallas as pl
from jax.experimental.pallas import tpu as pltpu

_B = 262144
_DIM = 6
_HALF = 3
_K = 40
_H = 128
_NL = 8
_PI = float(np.pi)
_BR = 4096

_ATAN_C = (0.9999999987329571, -0.3333329490271314, 0.19998530422323615,
           -0.14264510232090435, 0.10954998354223984, -0.0841450751516909,
           0.05818360636033609, -0.03143228778537418, 0.011064244656339386,
           -0.0018295627827675104)


def _atan(t):
    a = jnp.abs(t)
    big = a > 1.0
    r = jnp.where(big, 1.0 / a, a)
    r2 = r * r
    p = jnp.full_like(r2, _ATAN_C[-1])
    for c in _ATAN_C[-2::-1]:
        p = p * r2 + c
    at = p * r
    at = jnp.where(big, (_PI / 2.0) - at, at)
    return jnp.where(t < 0.0, -at, at)


def _fused_body(x_ref, Aeff_ref, b1_ref, W2T_ref, b2_ref, W3T_ref, b3_ref,
                ck_ref, R4_ref, M8_ref, o_ref):
    f32 = jnp.float32
    xT = jnp.transpose(x_ref[...])
    xpT = jnp.clip(xT[:_HALF], -1.0, 1.0)
    vT = xT[_HALF:] * (1.0 / _PI)
    n2 = jnp.sum(vT * vT, axis=0, keepdims=True)
    den = jax.lax.rsqrt(jnp.maximum(1.0 - n2, 1e-12))
    ycT = _atan(vT * den) * (2.0 / _PI)
    ycT = jnp.where(n2 < 1.0, ycT, 0.0)
    ypT = jnp.concatenate([xpT, ycT], axis=0)

    for i in range(_NL):
        a = jnp.dot(Aeff_ref[i], ypT, preferred_element_type=f32)
        h = jnp.maximum(a[:_H] + b1_ref[i], 0.0)
        h = jnp.maximum(
            jnp.dot(W2T_ref[i], h, preferred_element_type=f32) + b2_ref[i],
            0.0)
        th = jnp.dot(W3T_ref[i], h, preferred_element_type=f32) + b3_ref[i]
        e = jnp.exp(jnp.clip(th, -60.0, 60.0))
        w = jnp.clip(a[_H:2 * _H] + ck_ref[...], 0.0, 1.0)
        ewe = jnp.concatenate([e * w, e], axis=0)
        f4 = jnp.dot(R4_ref[...], ewe, preferred_element_type=f32)
        yt = f4[:_HALF] / f4[_HALF:]
        ypT = jnp.concatenate([a[2 * _H:2 * _H + _HALF], yt], axis=0)
    o_ref[...] = jax.lax.dot_general(
        ypT, M8_ref[...], (((0,), (0,)), ((), ())),
        preferred_element_type=f32)


@jax.jit
def kernel(x, W1, b1, W2, b2, W3, b3, orders, perms):
    f32 = jnp.float32
    eye6 = jnp.eye(_DIM, dtype=f32)
    inv = jnp.argsort(orders, axis=1)
    perm_mats = [jnp.transpose(eye6[orders[0]])]
    for i in range(_NL - 1):
        c = inv[i][perms[i]][orders[i + 1]]
        perm_mats.append(jnp.transpose(eye6[c]))
    perm_mats.append(jnp.transpose(eye6[inv[_NL - 1]]))

    gidx = np.repeat(np.arange(_HALF), _K)
    klocal = np.tile(np.arange(_K), _HALF).astype(np.float32)
    E20 = np.zeros((_HALF, _H), dtype=np.float32)
    E20[gidx, np.arange(_HALF * _K)] = 20.0
    E20 = jnp.asarray(E20)
    Aeff_list = []
    for i in range(_NL):
        M = perm_mats[i]
        blk1 = jnp.transpose(M[:, :_HALF] @ W1[i])
        blk2 = jnp.transpose(M[:, _HALF:] @ E20)
        blk3 = jnp.transpose(M[:, :_HALF])
        Aeff_list.append(jnp.concatenate(
            [blk1, blk2, blk3, jnp.zeros((5, _DIM), f32)], axis=0))
    Aeff = jnp.stack(Aeff_list)

    b1c = b1[:, :, None]
    b2c = b2[:, :, None]
    b3c = jnp.pad(b3, ((0, 0), (0, _H - _HALF * _K)))[:, :, None]
    ck = np.full((_H, 1), -1.0, dtype=np.float32)
    ck[:_HALF * _K, 0] = 20.0 - klocal
    ck = jnp.asarray(ck)

    W2T = jnp.transpose(W2, (0, 2, 1))
    W3T = jnp.transpose(
        jnp.pad(W3, ((0, 0), (0, 0), (0, _H - _HALF * _K))), (0, 2, 1))

    R4 = np.zeros((_DIM, 2 * _H), dtype=np.float32)
    R4[gidx, np.arange(_HALF * _K)] = 2.0
    R4[gidx, _H + np.arange(_HALF * _K)] = -1.0
    R4[_HALF + gidx, _H + np.arange(_HALF * _K)] = 1.0
    R4 = jnp.asarray(R4)

    M8 = perm_mats[_NL]

    grid = (_B // _BR,)
    out = pl.pallas_call(
        _fused_body,
        grid=grid,
        in_specs=[
            pl.BlockSpec((_BR, _DIM), lambda i: (i, 0)),
            pl.BlockSpec((_NL, 264, _DIM), lambda i: (0, 0, 0)),
            pl.BlockSpec((_NL, _H, 1), lambda i: (0, 0, 0)),
            pl.BlockSpec((_NL, _H, _H), lambda i: (0, 0, 0)),
            pl.BlockSpec((_NL, _H, 1), lambda i: (0, 0, 0)),
            pl.BlockSpec((_NL, _H, _H), lambda i: (0, 0, 0)),
            pl.BlockSpec((_NL, _H, 1), lambda i: (0, 0, 0)),
            pl.BlockSpec((_H, 1), lambda i: (0, 0)),
            pl.BlockSpec((_DIM, 2 * _H), lambda i: (0, 0)),
            pl.BlockSpec((_DIM, _DIM), lambda i: (0, 0)),
        ],
        out_specs=pl.BlockSpec((_BR, _DIM), lambda i: (i, 0)),
        out_shape=jax.ShapeDtypeStruct((_B, _DIM), f32),
        compiler_params=pltpu.CompilerParams(
            dimension_semantics=("parallel",)),
    )(x, Aeff, b1c, W2T, b2c, W3T, b3c, ck, R4, M8)
    return out


# speedup vs baseline: 10.2377x; 1.0114x over previous
import jax
import jax.numpy as jnp
import numpy as np
from jax.experimental import pallas as pl
from jax.experimental.pallas import tpu as pltpu

_B = 262144
_DIM = 6
_HALF = 3
_K = 40
_H = 128
_NL = 8
_PI = float(np.pi)
_BR = 4096

_ATAN_C = (0.9999999987329571, -0.3333329490271314, 0.19998530422323615,
           -0.14264510232090435, 0.10954998354223984, -0.0841450751516909,
           0.05818360636033609, -0.03143228778537418, 0.011064244656339386,
           -0.0018295627827675104)


def _atan(t):
    a = jnp.abs(t)
    big = a > 1.0
    r = jnp.where(big, 1.0 / a, a)
    r2 = r * r
    p = jnp.full_like(r2, _ATAN_C[-1])
    for c in _ATAN_C[-2::-1]:
        p = p * r2 + c
    at = p * r
    at = jnp.where(big, (_PI / 2.0) - at, at)
    return jnp.where(t < 0.0, -at, at)


def _fused_body(x_ref, Aeff_ref, b1_ref, W2T_ref, b2_ref, W3T_ref, b3_ref,
                ck_ref, R4a_ref, R4b_ref, M8_ref, o_ref):
    f32 = jnp.float32
    xT = jnp.transpose(x_ref[...])
    xpT = jnp.clip(xT[:_HALF], -1.0, 1.0)
    vT = xT[_HALF:] * (1.0 / _PI)
    n2 = jnp.sum(vT * vT, axis=0, keepdims=True)
    den = jax.lax.rsqrt(jnp.maximum(1.0 - n2, 1e-12))
    ycT = _atan(vT * den) * (2.0 / _PI)
    ycT = jnp.where(n2 < 1.0, ycT, 0.0)
    ypT = jnp.concatenate([xpT, ycT], axis=0)

    for i in range(_NL):
        a = jnp.dot(Aeff_ref[i], ypT, preferred_element_type=f32)
        h = jnp.maximum(a[:_H] + b1_ref[i], 0.0).astype(jnp.bfloat16)
        h = jnp.maximum(
            jnp.dot(W2T_ref[i], h, preferred_element_type=f32) + b2_ref[i],
            0.0).astype(jnp.bfloat16)
        th = jnp.dot(W3T_ref[i], h, preferred_element_type=f32) + b3_ref[i]
        e = jnp.exp(jnp.clip(th, -60.0, 60.0))
        w = jnp.clip(a[_H:2 * _H] + ck_ref[...], 0.0, 1.0)
        f4 = (jnp.dot(R4a_ref[...], e * w, preferred_element_type=f32)
              + jnp.dot(R4b_ref[...], e, preferred_element_type=f32))
        yt = f4[:_HALF] / f4[_HALF:]
        ypT = jnp.concatenate([a[2 * _H:2 * _H + _HALF], yt], axis=0)
    o_ref[...] = jax.lax.dot_general(
        ypT, M8_ref[...], (((0,), (0,)), ((), ())),
        preferred_element_type=f32)


@jax.jit
def kernel(x, W1, b1, W2, b2, W3, b3, orders, perms):
    f32 = jnp.float32
    eye6 = jnp.eye(_DIM, dtype=f32)
    inv = jnp.argsort(orders, axis=1)
    perm_mats = [jnp.transpose(eye6[orders[0]])]
    for i in range(_NL - 1):
        c = inv[i][perms[i]][orders[i + 1]]
        perm_mats.append(jnp.transpose(eye6[c]))
    perm_mats.append(jnp.transpose(eye6[inv[_NL - 1]]))

    gidx = np.repeat(np.arange(_HALF), _K)
    klocal = np.tile(np.arange(_K), _HALF).astype(np.float32)
    E20 = np.zeros((_HALF, _H), dtype=np.float32)
    E20[gidx, np.arange(_HALF * _K)] = 20.0
    E20 = jnp.asarray(E20)
    Aeff_list = []
    for i in range(_NL):
        M = perm_mats[i]
        blk1 = jnp.transpose(M[:, :_HALF] @ W1[i])
        blk2 = jnp.transpose(M[:, _HALF:] @ E20)
        blk3 = jnp.transpose(M[:, :_HALF])
        Aeff_list.append(jnp.concatenate(
            [blk1, blk2, blk3, jnp.zeros((5, _DIM), f32)], axis=0))
    Aeff = jnp.stack(Aeff_list)

    b1c = b1[:, :, None]
    b2c = b2[:, :, None]
    b3c = jnp.pad(b3, ((0, 0), (0, _H - _HALF * _K)))[:, :, None]
    ck = np.full((_H, 1), -1.0, dtype=np.float32)
    ck[:_HALF * _K, 0] = 20.0 - klocal
    ck = jnp.asarray(ck)

    W2T = jnp.transpose(W2, (0, 2, 1)).astype(jnp.bfloat16)
    W3T = jnp.transpose(
        jnp.pad(W3, ((0, 0), (0, 0), (0, _H - _HALF * _K))),
        (0, 2, 1)).astype(jnp.bfloat16)

    R4a = np.zeros((_DIM, _H), dtype=np.float32)
    R4a[gidx, np.arange(_HALF * _K)] = 2.0
    R4b = np.zeros((_DIM, _H), dtype=np.float32)
    R4b[gidx, np.arange(_HALF * _K)] = -1.0
    R4b[_HALF + gidx, np.arange(_HALF * _K)] = 1.0
    R4a = jnp.asarray(R4a)
    R4b = jnp.asarray(R4b)

    M8 = perm_mats[_NL]

    grid = (_B // _BR,)
    out = pl.pallas_call(
        _fused_body,
        grid=grid,
        in_specs=[
            pl.BlockSpec((_BR, _DIM), lambda i: (i, 0)),
            pl.BlockSpec((_NL, 264, _DIM), lambda i: (0, 0, 0)),
            pl.BlockSpec((_NL, _H, 1), lambda i: (0, 0, 0)),
            pl.BlockSpec((_NL, _H, _H), lambda i: (0, 0, 0)),
            pl.BlockSpec((_NL, _H, 1), lambda i: (0, 0, 0)),
            pl.BlockSpec((_NL, _H, _H), lambda i: (0, 0, 0)),
            pl.BlockSpec((_NL, _H, 1), lambda i: (0, 0, 0)),
            pl.BlockSpec((_H, 1), lambda i: (0, 0)),
            pl.BlockSpec((_DIM, _H), lambda i: (0, 0)),
            pl.BlockSpec((_DIM, _H), lambda i: (0, 0)),
            pl.BlockSpec((_DIM, _DIM), lambda i: (0, 0)),
        ],
        out_specs=pl.BlockSpec((_BR, _DIM), lambda i: (i, 0)),
        out_shape=jax.ShapeDtypeStruct((_B, _DIM), f32),
        compiler_params=pltpu.CompilerParams(
            dimension_semantics=("parallel",)),
    )(x, Aeff, b1c, W2T, b2c, W3T, b3c, ck, R4a, R4b, M8)
    return out
